# baseline (speedup 1.0000x reference)
import functools

import jax
import jax.numpy as jnp
from jax import lax
from jax.experimental import pallas as pl
from jax.experimental.pallas import tpu as pltpu

IN_F = 4096
OUT_F = 4096
N_EXP = 8
PACK = 8
N_TOK = 32
BLK_O = 1024
K_SLC = IN_F // N_EXP
QCOL = IN_F // PACK
HCOL = 2 * QCOL
NCHUNK = PACK // 2

_NIBMASK = 0x000F000F
_MAGIC = 0x43004300


def _body(x_ref, xq_ref, idx_ref, w_ref, q_ref, rs_ref, z_ref, s_ref, o_ref):
    e = pl.program_id(1)
    mask = idx_ref[...] == e

    qq = pltpu.bitcast(q_ref[0], jnp.int32)
    dot = jnp.zeros((N_TOK, BLK_O), jnp.float32)
    for n in range(NCHUNK):
        t = ((qq >> (4 * n)) & _NIBMASK) | _MAGIC
        u = pltpu.bitcast(t, jnp.bfloat16) - jnp.bfloat16(128)
        xm = jnp.where(mask, xq_ref[n], jnp.bfloat16(0))
        dot += lax.dot_general(
            xm, u, (((1,), (1,)), ((), ())),
            preferred_element_type=jnp.float32,
        )
    delta = (dot - rs_ref[0] * z_ref[0]) * s_ref[0]

    wb = w_ref[...].astype(jnp.bfloat16)
    base = lax.dot_general(
        x_ref[...], wb, (((1,), (1,)), ((), ())),
        preferred_element_type=jnp.float32,
    )

    @pl.when(e == 0)
    def _():
        o_ref[...] = base + delta

    @pl.when(e != 0)
    def _():
        o_ref[...] += base + delta


@jax.jit
def _run(x, xq, idx, weight, qweight16, rs, z, s):
    grid = (OUT_F // BLK_O, N_EXP)
    return pl.pallas_call(
        _body,
        grid=grid,
        in_specs=[
            pl.BlockSpec((N_TOK, K_SLC), lambda o, e: (0, e)),
            pl.BlockSpec((NCHUNK, N_TOK, HCOL), lambda o, e: (0, 0, 0)),
            pl.BlockSpec((N_TOK, 1), lambda o, e: (0, 0)),
            pl.BlockSpec((BLK_O, K_SLC), lambda o, e: (o, e)),
            pl.BlockSpec((1, BLK_O, HCOL), lambda o, e: (e, o, 0)),
            pl.BlockSpec((1, N_TOK, 1), lambda o, e: (e, 0, 0)),
            pl.BlockSpec((1, 1, BLK_O), lambda o, e: (e, 0, o)),
            pl.BlockSpec((1, 1, BLK_O), lambda o, e: (e, 0, o)),
        ],
        out_specs=pl.BlockSpec((N_TOK, BLK_O), lambda o, e: (0, o)),
        out_shape=jax.ShapeDtypeStruct((N_TOK, OUT_F), jnp.float32),
        compiler_params=pltpu.CompilerParams(
            dimension_semantics=("parallel", "arbitrary"),
        ),
    )(x, xq, idx, weight, qweight16, rs, z, s)


def kernel(input_, weight, scales_stacked, qweight_stacked, qzeros_stacked, indices):
    x = input_.astype(jnp.bfloat16)
    qweight16 = lax.bitcast_convert_type(qweight_stacked, jnp.int16).reshape(
        N_EXP, OUT_F, HCOL
    )
    xq = (
        input_.reshape(N_TOK, QCOL, 2, NCHUNK)
        .transpose(3, 0, 1, 2)
        .reshape(NCHUNK, N_TOK, HCOL)
        .astype(jnp.bfloat16)
    )
    idx = indices.reshape(N_TOK, 1)
    qz = qzeros_stacked.reshape(N_EXP, OUT_F // PACK)
    shifts = jnp.arange(PACK, dtype=jnp.int32) * 4
    z = ((qz[:, :, None] >> shifts) & 15).astype(jnp.float32).reshape(
        N_EXP, 1, OUT_F
    )
    s = scales_stacked.reshape(N_EXP, 1, OUT_F)
    xsum = jnp.sum(x.astype(jnp.float32), axis=1)
    onehot = (indices[None, :] == jnp.arange(N_EXP, dtype=jnp.int32)[:, None])
    rs = (onehot * xsum[None, :]).astype(jnp.float32).reshape(N_EXP, N_TOK, 1)
    return _run(x, xq, idx, weight, qweight16, rs, z, s)


# speedup vs baseline: 3.7413x; 3.7413x over previous
import functools

import jax
import jax.numpy as jnp
from jax import lax
from jax.experimental import pallas as pl
from jax.experimental.pallas import tpu as pltpu

IN_F = 4096
OUT_F = 4096
N_EXP = 8
PACK = 8
N_TOK = 32
BLK_O = 512
K_SLC = IN_F // N_EXP


def _body(x_ref, xp_ref, idx_ref, w_ref, q_ref, z_ref, s_ref, o_ref):
    e = pl.program_id(1)

    q = q_ref[0]
    parts = [q & 15]
    for n in range(1, PACK - 1):
        parts.append((q >> (4 * n)) & 15)
    parts.append(q >> (4 * (PACK - 1)))
    u = jnp.concatenate(parts, axis=1).astype(jnp.bfloat16)

    mask = idx_ref[...] == e
    xm = jnp.where(mask, xp_ref[...], jnp.bfloat16(0))
    dot = lax.dot_general(
        xm, u, (((1,), (1,)), ((), ())), preferred_element_type=jnp.float32
    )
    rs = jnp.sum(xm.astype(jnp.float32), axis=1, keepdims=True)
    delta = (dot - rs * z_ref[0]) * s_ref[0]

    wb = w_ref[...].astype(jnp.bfloat16)
    base = lax.dot_general(
        x_ref[...], wb, (((1,), (1,)), ((), ())),
        preferred_element_type=jnp.float32,
    )

    @pl.when(e == 0)
    def _():
        o_ref[...] = base + delta

    @pl.when(e != 0)
    def _():
        o_ref[...] += base + delta


@jax.jit
def _run(x, xp, idx, weight, qweight, z, s):
    grid = (OUT_F // BLK_O, N_EXP)
    return pl.pallas_call(
        _body,
        grid=grid,
        in_specs=[
            pl.BlockSpec((N_TOK, K_SLC), lambda o, e: (0, e)),
            pl.BlockSpec((N_TOK, IN_F), lambda o, e: (0, 0)),
            pl.BlockSpec((N_TOK, 1), lambda o, e: (0, 0)),
            pl.BlockSpec((BLK_O, K_SLC), lambda o, e: (o, e)),
            pl.BlockSpec((1, BLK_O, IN_F // PACK), lambda o, e: (e, o, 0)),
            pl.BlockSpec((1, 1, BLK_O), lambda o, e: (e, 0, o)),
            pl.BlockSpec((1, 1, BLK_O), lambda o, e: (e, 0, o)),
        ],
        out_specs=pl.BlockSpec((N_TOK, BLK_O), lambda o, e: (0, o)),
        out_shape=jax.ShapeDtypeStruct((N_TOK, OUT_F), jnp.float32),
        compiler_params=pltpu.CompilerParams(
            dimension_semantics=("parallel", "arbitrary"),
        ),
    )(x, xp, idx, weight, qweight, z, s)


def kernel(input_, weight, scales_stacked, qweight_stacked, qzeros_stacked, indices):
    x = input_.astype(jnp.bfloat16)
    xp = (
        input_.reshape(N_TOK, IN_F // PACK, PACK)
        .transpose(0, 2, 1)
        .reshape(N_TOK, IN_F)
        .astype(jnp.bfloat16)
    )
    idx = indices.reshape(N_TOK, 1)
    qz = qzeros_stacked.reshape(N_EXP, OUT_F // PACK)
    shifts = jnp.arange(PACK, dtype=jnp.int32) * 4
    z = ((qz[:, :, None] >> shifts) & 15).astype(jnp.float32).reshape(
        N_EXP, 1, OUT_F
    )
    s = scales_stacked.reshape(N_EXP, 1, OUT_F)
    return _run(x, xp, idx, weight, qweight_stacked, z, s)


# speedup vs baseline: 3.9600x; 1.0585x over previous
import functools

import jax
import jax.numpy as jnp
from jax import lax
from jax.experimental import pallas as pl
from jax.experimental.pallas import tpu as pltpu

IN_F = 4096
OUT_F = 4096
N_EXP = 8
PACK = 8
N_TOK = 32
BLK_O = 512
QCOL = IN_F // PACK


def _unpack_bf16(q):
    parts = [q & 15]
    for n in range(1, PACK - 1):
        parts.append((q >> (4 * n)) & 15)
    parts.append(q >> (4 * (PACK - 1)))
    return jnp.concatenate(parts, axis=1).astype(jnp.bfloat16)


def _body(x_ref, xp_ref, idx_ref, w_ref, q_ref, rs_ref, z_ref, s_ref, o_ref):
    wb = w_ref[...].astype(jnp.bfloat16)
    acc = lax.dot_general(
        x_ref[...], wb, (((1,), (1,)), ((), ())),
        preferred_element_type=jnp.float32,
    )
    for e in range(N_EXP):
        u = _unpack_bf16(q_ref[e])
        xm = jnp.where(idx_ref[...] == e, xp_ref[...], jnp.bfloat16(0))
        dot = lax.dot_general(
            xm, u, (((1,), (1,)), ((), ())),
            preferred_element_type=jnp.float32,
        )
        acc += (dot - rs_ref[0, :, e:e + 1] * z_ref[e]) * s_ref[e]
    o_ref[...] = acc


@jax.jit
def _run(x, xp, idx, weight, qweight, rs, z, s):
    grid = (OUT_F // BLK_O,)
    return pl.pallas_call(
        _body,
        grid=grid,
        in_specs=[
            pl.BlockSpec((N_TOK, IN_F), lambda o: (0, 0)),
            pl.BlockSpec((N_TOK, IN_F), lambda o: (0, 0)),
            pl.BlockSpec((N_TOK, 1), lambda o: (0, 0)),
            pl.BlockSpec((BLK_O, IN_F), lambda o: (o, 0)),
            pl.BlockSpec((N_EXP, BLK_O, QCOL), lambda o: (0, o, 0)),
            pl.BlockSpec((1, N_TOK, N_EXP), lambda o: (0, 0, 0)),
            pl.BlockSpec((N_EXP, 1, BLK_O), lambda o: (0, 0, o)),
            pl.BlockSpec((N_EXP, 1, BLK_O), lambda o: (0, 0, o)),
        ],
        out_specs=pl.BlockSpec((N_TOK, BLK_O), lambda o: (0, o)),
        out_shape=jax.ShapeDtypeStruct((N_TOK, OUT_F), jnp.float32),
        compiler_params=pltpu.CompilerParams(
            dimension_semantics=("arbitrary",),
        ),
    )(x, xp, idx, weight, qweight, rs, z, s)


def kernel(input_, weight, scales_stacked, qweight_stacked, qzeros_stacked, indices):
    x = input_.astype(jnp.bfloat16)
    xp = (
        input_.reshape(N_TOK, QCOL, PACK)
        .transpose(0, 2, 1)
        .reshape(N_TOK, IN_F)
        .astype(jnp.bfloat16)
    )
    idx = indices.reshape(N_TOK, 1)
    qz = qzeros_stacked.reshape(N_EXP, OUT_F // PACK)
    shifts = jnp.arange(PACK, dtype=jnp.int32) * 4
    z = ((qz[:, :, None] >> shifts) & 15).astype(jnp.float32).reshape(
        N_EXP, 1, OUT_F
    )
    s = scales_stacked.reshape(N_EXP, 1, OUT_F)
    xsum = jnp.sum(xp.astype(jnp.float32), axis=1)
    onehot = indices[:, None] == jnp.arange(N_EXP, dtype=jnp.int32)[None, :]
    rs = (onehot * xsum[:, None]).astype(jnp.float32).reshape(1, N_TOK, N_EXP)
    return _run(x, xp, idx, weight, qweight_stacked, rs, z, s)
